# baseline (speedup 1.0000x reference)
import jax
import jax.numpy as jnp
from jax.experimental import pallas as pl
from jax.experimental.pallas import tpu as pltpu

B = 4096
BLK = 256
NBLK = B // BLK


def _fused_kernel(sensor_s_ref, sensor_ref, other_s_ref, other_ref,
                  act_p_ref, sens_p_ref, act_ref, sens_ref,
                  out_ref, acc_ref):
    i = pl.program_id(0)

    @pl.when(i == 0)
    def _init():
        for k in range(16):
            acc_ref[k] = 0.0

    d = jnp.abs(sensor_s_ref[...] - sensor_ref[...])
    acc_ref[0] += jnp.sum(d)
    o = jnp.abs(other_s_ref[...] - other_ref[...])
    acc_ref[1] += jnp.sum(o)

    ap = act_p_ref[...]
    sp = sens_p_ref[...]
    ar = act_ref[...]
    sr = sens_ref[...]
    na = ap.shape[1]
    ns = sp.shape[1]
    iota_a = jax.lax.broadcasted_iota(jnp.int32, (ap.shape[0], na), 1)
    iota_s = jax.lax.broadcasted_iota(jnp.int32, (sp.shape[0], ns), 1)
    va = jnp.abs(1.0 - jnp.sum(jnp.where(iota_a == ar, ap, 0.0), axis=1,
                               keepdims=True))
    vs = jnp.abs(1.0 - jnp.sum(jnp.where(iota_s == sr, sp, 0.0), axis=1,
                               keepdims=True))

    for g in range(4):
        mg = (sr == g)
        acc_ref[2 + g] += jnp.sum(jnp.where(mg, va, 0.0))
        acc_ref[6 + g] += jnp.sum(jnp.where(mg, vs, 0.0))
        acc_ref[10 + g] += jnp.sum(mg.astype(jnp.float32))
    acc_ref[14] = jnp.maximum(acc_ref[14], jnp.max(sr).astype(jnp.float32))

    @pl.when(i == NBLK - 1)
    def _final():
        n_groups = acc_ref[14] + 1.0
        s_act = 0.0
        s_sens = 0.0
        for g in range(4):
            cnt = jnp.maximum(acc_ref[10 + g], 1e-12)
            s_act = s_act + acc_ref[2 + g] / cnt
            s_sens = s_sens + acc_ref[6 + g] / cnt
        act_loss = jnp.abs(0.0 - s_act / n_groups)
        sens_loss = jnp.abs(0.5 - s_sens / n_groups)
        sensor_loss = acc_ref[0] / (4096.0 * 6.0 * 512.0)
        physio_loss = acc_ref[1] / (4096.0 * 16.0)
        san_mean = 0.5 * (sensor_loss + physio_loss)
        combined = 0.25 * act_loss + 0.25 * sens_loss + 0.5 * san_mean
        out_ref[0] = combined
        out_ref[1] = act_loss
        out_ref[2] = sens_loss


def kernel(sensor_s, other_s, act_p, sens_p, sensor, act, sens, other):
    act_col = act.reshape(B, 1)
    sens_col = sens.reshape(B, 1)
    ct = sensor_s.shape[1] * sensor_s.shape[2]
    s2 = sensor_s.reshape(B, ct)
    r2 = sensor.reshape(B, ct)

    out = pl.pallas_call(
        _fused_kernel,
        grid=(NBLK,),
        in_specs=[
            pl.BlockSpec((BLK, ct), lambda i: (i, 0)),
            pl.BlockSpec((BLK, ct), lambda i: (i, 0)),
            pl.BlockSpec((BLK, other_s.shape[1]), lambda i: (i, 0)),
            pl.BlockSpec((BLK, other.shape[1]), lambda i: (i, 0)),
            pl.BlockSpec((BLK, act_p.shape[1]), lambda i: (i, 0)),
            pl.BlockSpec((BLK, sens_p.shape[1]), lambda i: (i, 0)),
            pl.BlockSpec((BLK, 1), lambda i: (i, 0)),
            pl.BlockSpec((BLK, 1), lambda i: (i, 0)),
        ],
        out_specs=pl.BlockSpec(memory_space=pltpu.SMEM),
        out_shape=jax.ShapeDtypeStruct((4,), jnp.float32),
        scratch_shapes=[pltpu.SMEM((16,), jnp.float32)],
    )(s2, r2, other_s, other, act_p, sens_p, act_col, sens_col)

    return (out[0], out[1], out[2])


# speedup vs baseline: 1.3749x; 1.3749x over previous
import jax
import jax.numpy as jnp
from jax.experimental import pallas as pl
from jax.experimental.pallas import tpu as pltpu

B = 4096
BLK = 512
NBLK = B // BLK


def _fused_kernel(sensor_s_ref, sensor_ref, other_s_ref, other_ref,
                  act_p_ref, sens_p_ref, act_ref, sens_ref,
                  out_ref, acc_ref):
    i = pl.program_id(0)

    @pl.when(i == 0)
    def _init():
        for k in range(16):
            acc_ref[k] = 0.0

    d = jnp.abs(sensor_s_ref[...] - sensor_ref[...])
    acc_ref[0] += jnp.sum(d)
    o = jnp.abs(other_s_ref[...] - other_ref[...])
    acc_ref[1] += jnp.sum(o)

    ap = act_p_ref[...]
    sp = sens_p_ref[...]
    ar = act_ref[...]
    sr = sens_ref[...]
    na = ap.shape[1]
    ns = sp.shape[1]
    iota_a = jax.lax.broadcasted_iota(jnp.int32, (ap.shape[0], na), 1)
    iota_s = jax.lax.broadcasted_iota(jnp.int32, (sp.shape[0], ns), 1)
    va = jnp.abs(1.0 - jnp.sum(jnp.where(iota_a == ar, ap, 0.0), axis=1,
                               keepdims=True))
    vs = jnp.abs(1.0 - jnp.sum(jnp.where(iota_s == sr, sp, 0.0), axis=1,
                               keepdims=True))

    for g in range(4):
        mg = (sr == g)
        acc_ref[2 + g] += jnp.sum(jnp.where(mg, va, 0.0))
        acc_ref[6 + g] += jnp.sum(jnp.where(mg, vs, 0.0))
        acc_ref[10 + g] += jnp.sum(mg.astype(jnp.float32))
    acc_ref[14] = jnp.maximum(acc_ref[14], jnp.max(sr).astype(jnp.float32))

    @pl.when(i == NBLK - 1)
    def _final():
        n_groups = acc_ref[14] + 1.0
        s_act = 0.0
        s_sens = 0.0
        for g in range(4):
            cnt = jnp.maximum(acc_ref[10 + g], 1e-12)
            s_act = s_act + acc_ref[2 + g] / cnt
            s_sens = s_sens + acc_ref[6 + g] / cnt
        act_loss = jnp.abs(0.0 - s_act / n_groups)
        sens_loss = jnp.abs(0.5 - s_sens / n_groups)
        sensor_loss = acc_ref[0] / (4096.0 * 6.0 * 512.0)
        physio_loss = acc_ref[1] / (4096.0 * 16.0)
        san_mean = 0.5 * (sensor_loss + physio_loss)
        combined = 0.25 * act_loss + 0.25 * sens_loss + 0.5 * san_mean
        out_ref[0] = combined
        out_ref[1] = act_loss
        out_ref[2] = sens_loss


def kernel(sensor_s, other_s, act_p, sens_p, sensor, act, sens, other):
    act_col = act.reshape(B, 1)
    sens_col = sens.reshape(B, 1)
    c, t = sensor_s.shape[1], sensor_s.shape[2]

    out = pl.pallas_call(
        _fused_kernel,
        grid=(NBLK,),
        in_specs=[
            pl.BlockSpec((BLK, c, t), lambda i: (i, 0, 0)),
            pl.BlockSpec((BLK, c, t), lambda i: (i, 0, 0)),
            pl.BlockSpec((BLK, other_s.shape[1]), lambda i: (i, 0)),
            pl.BlockSpec((BLK, other.shape[1]), lambda i: (i, 0)),
            pl.BlockSpec((BLK, act_p.shape[1]), lambda i: (i, 0)),
            pl.BlockSpec((BLK, sens_p.shape[1]), lambda i: (i, 0)),
            pl.BlockSpec((BLK, 1), lambda i: (i, 0)),
            pl.BlockSpec((BLK, 1), lambda i: (i, 0)),
        ],
        out_specs=pl.BlockSpec(memory_space=pltpu.SMEM),
        out_shape=jax.ShapeDtypeStruct((4,), jnp.float32),
        scratch_shapes=[pltpu.SMEM((16,), jnp.float32)],
    )(sensor_s, sensor, other_s, other, act_p, sens_p, act_col, sens_col)

    return (out[0], out[1], out[2])


# speedup vs baseline: 1.3778x; 1.0021x over previous
import jax
import jax.numpy as jnp
from jax.experimental import pallas as pl
from jax.experimental.pallas import tpu as pltpu

B = 4096
BLK = 256
NBLK = B // BLK


def _fused_kernel(sensor_s_ref, sensor_ref, other_s_ref, other_ref,
                  act_p_ref, sens_p_ref, act_ref, sens_ref,
                  out_ref, acc_ref):
    i = pl.program_id(0)

    @pl.when(i == 0)
    def _init():
        for k in range(16):
            acc_ref[k] = 0.0

    d = jnp.abs(sensor_s_ref[...] - sensor_ref[...])
    acc_ref[0] += jnp.sum(d)
    o = jnp.abs(other_s_ref[...] - other_ref[...])
    acc_ref[1] += jnp.sum(o)

    ap = act_p_ref[...]
    sp = sens_p_ref[...]
    ar = act_ref[...]
    sr = sens_ref[...]
    na = ap.shape[1]
    ns = sp.shape[1]
    iota_a = jax.lax.broadcasted_iota(jnp.int32, (ap.shape[0], na), 1)
    iota_s = jax.lax.broadcasted_iota(jnp.int32, (sp.shape[0], ns), 1)
    va = jnp.abs(1.0 - jnp.sum(jnp.where(iota_a == ar, ap, 0.0), axis=1,
                               keepdims=True))
    vs = jnp.abs(1.0 - jnp.sum(jnp.where(iota_s == sr, sp, 0.0), axis=1,
                               keepdims=True))

    for g in range(4):
        mg = (sr == g)
        acc_ref[2 + g] += jnp.sum(jnp.where(mg, va, 0.0))
        acc_ref[6 + g] += jnp.sum(jnp.where(mg, vs, 0.0))
        acc_ref[10 + g] += jnp.sum(mg.astype(jnp.float32))
    acc_ref[14] = jnp.maximum(acc_ref[14], jnp.max(sr).astype(jnp.float32))

    @pl.when(i == NBLK - 1)
    def _final():
        n_groups = acc_ref[14] + 1.0
        s_act = 0.0
        s_sens = 0.0
        for g in range(4):
            cnt = jnp.maximum(acc_ref[10 + g], 1e-12)
            s_act = s_act + acc_ref[2 + g] / cnt
            s_sens = s_sens + acc_ref[6 + g] / cnt
        act_loss = jnp.abs(0.0 - s_act / n_groups)
        sens_loss = jnp.abs(0.5 - s_sens / n_groups)
        sensor_loss = acc_ref[0] / (4096.0 * 6.0 * 512.0)
        physio_loss = acc_ref[1] / (4096.0 * 16.0)
        san_mean = 0.5 * (sensor_loss + physio_loss)
        combined = 0.25 * act_loss + 0.25 * sens_loss + 0.5 * san_mean
        out_ref[0] = combined
        out_ref[1] = act_loss
        out_ref[2] = sens_loss


def kernel(sensor_s, other_s, act_p, sens_p, sensor, act, sens, other):
    act_col = act.reshape(B, 1)
    sens_col = sens.reshape(B, 1)
    c, t = sensor_s.shape[1], sensor_s.shape[2]

    out = pl.pallas_call(
        _fused_kernel,
        grid=(NBLK,),
        in_specs=[
            pl.BlockSpec((BLK, c, t), lambda i: (i, 0, 0)),
            pl.BlockSpec((BLK, c, t), lambda i: (i, 0, 0)),
            pl.BlockSpec((BLK, other_s.shape[1]), lambda i: (i, 0)),
            pl.BlockSpec((BLK, other.shape[1]), lambda i: (i, 0)),
            pl.BlockSpec((BLK, act_p.shape[1]), lambda i: (i, 0)),
            pl.BlockSpec((BLK, sens_p.shape[1]), lambda i: (i, 0)),
            pl.BlockSpec((BLK, 1), lambda i: (i, 0)),
            pl.BlockSpec((BLK, 1), lambda i: (i, 0)),
        ],
        out_specs=pl.BlockSpec(memory_space=pltpu.SMEM),
        out_shape=jax.ShapeDtypeStruct((4,), jnp.float32),
        scratch_shapes=[pltpu.SMEM((16,), jnp.float32)],
    )(sensor_s, sensor, other_s, other, act_p, sens_p, act_col, sens_col)

    return (out[0], out[1], out[2])


# speedup vs baseline: 6.1755x; 4.4820x over previous
import jax
import jax.numpy as jnp
from jax.experimental import pallas as pl
from jax.experimental.pallas import tpu as pltpu

B = 4096
BLK = 512
NBLK = B // BLK


def _fused_kernel(sensor_s_ref, sensor_ref, other_s_ref, other_ref,
                  act_p_ref, sens_p_ref, act_ref, sens_ref,
                  out_ref, acc_ref):
    i = pl.program_id(0)

    @pl.when(i == 0)
    def _init():
        for k in range(16):
            acc_ref[k] = 0.0

    d = jnp.abs(sensor_s_ref[...] - sensor_ref[...])
    acc_ref[0] += jnp.sum(d)
    o = jnp.abs(other_s_ref[...] - other_ref[...])
    acc_ref[1] += jnp.sum(o)

    ap = act_p_ref[...]
    sp = sens_p_ref[...]
    ar = act_ref[...]
    sr = sens_ref[...]
    na = ap.shape[0]
    ns = sp.shape[0]
    iota_a = jax.lax.broadcasted_iota(jnp.int32, (na, ap.shape[1]), 0)
    iota_s = jax.lax.broadcasted_iota(jnp.int32, (ns, sp.shape[1]), 0)
    va = jnp.abs(1.0 - jnp.sum(jnp.where(iota_a == ar, ap, 0.0), axis=0,
                               keepdims=True))
    vs = jnp.abs(1.0 - jnp.sum(jnp.where(iota_s == sr, sp, 0.0), axis=0,
                               keepdims=True))

    for g in range(4):
        mg = (sr == g)
        acc_ref[2 + g] += jnp.sum(jnp.where(mg, va, 0.0))
        acc_ref[6 + g] += jnp.sum(jnp.where(mg, vs, 0.0))
        acc_ref[10 + g] += jnp.sum(mg.astype(jnp.float32))
    acc_ref[14] = jnp.maximum(acc_ref[14], jnp.max(sr).astype(jnp.float32))

    @pl.when(i == NBLK - 1)
    def _final():
        n_groups = acc_ref[14] + 1.0
        s_act = 0.0
        s_sens = 0.0
        for g in range(4):
            cnt = jnp.maximum(acc_ref[10 + g], 1e-12)
            s_act = s_act + acc_ref[2 + g] / cnt
            s_sens = s_sens + acc_ref[6 + g] / cnt
        act_loss = jnp.abs(0.0 - s_act / n_groups)
        sens_loss = jnp.abs(0.5 - s_sens / n_groups)
        sensor_loss = acc_ref[0] / (4096.0 * 6.0 * 512.0)
        physio_loss = acc_ref[1] / (4096.0 * 16.0)
        san_mean = 0.5 * (sensor_loss + physio_loss)
        combined = 0.25 * act_loss + 0.25 * sens_loss + 0.5 * san_mean
        out_ref[0] = combined
        out_ref[1] = act_loss
        out_ref[2] = sens_loss


def kernel(sensor_s, other_s, act_p, sens_p, sensor, act, sens, other):
    c, t = sensor_s.shape[1], sensor_s.shape[2]
    st = jnp.transpose(sensor_s, (1, 0, 2))
    rt = jnp.transpose(sensor, (1, 0, 2))
    ot_s = other_s.T
    ot = other.T
    apt = act_p.T
    spt = sens_p.T
    ar = act.reshape(1, B)
    sr = sens.reshape(1, B)

    out = pl.pallas_call(
        _fused_kernel,
        grid=(NBLK,),
        in_specs=[
            pl.BlockSpec((c, BLK, t), lambda i: (0, i, 0)),
            pl.BlockSpec((c, BLK, t), lambda i: (0, i, 0)),
            pl.BlockSpec((ot_s.shape[0], BLK), lambda i: (0, i)),
            pl.BlockSpec((ot.shape[0], BLK), lambda i: (0, i)),
            pl.BlockSpec((apt.shape[0], BLK), lambda i: (0, i)),
            pl.BlockSpec((spt.shape[0], BLK), lambda i: (0, i)),
            pl.BlockSpec((1, BLK), lambda i: (0, i)),
            pl.BlockSpec((1, BLK), lambda i: (0, i)),
        ],
        out_specs=pl.BlockSpec(memory_space=pltpu.SMEM),
        out_shape=jax.ShapeDtypeStruct((4,), jnp.float32),
        scratch_shapes=[pltpu.SMEM((16,), jnp.float32)],
    )(st, rt, ot_s, ot, apt, spt, ar, sr)

    return (out[0], out[1], out[2])
